# baseline (speedup 1.0000x reference)
import functools

import jax
import jax.numpy as jnp
from jax import lax
from jax.experimental import pallas as pl
from jax.experimental.pallas import tpu as pltpu
from jax.experimental.pallas import tpu_sc as plsc

B, T, H = 1024, 200, 768
POS_OFF = 2
N = B * T
NC, NS = 2, 16
NW = NC * NS
ROWS_W = N // NW
SEQ_W = ROWS_W // T
TCH = 40
NTC = T // TCH
VPR = H // 16

_mesh = plsc.VectorSubcoreMesh(core_axis_name="c", subcore_axis_name="s")


@functools.partial(
    pl.kernel,
    out_type=jax.ShapeDtypeStruct((N, H), jnp.float32),
    mesh=_mesh,
    scratch_types=[
        pltpu.VMEM((ROWS_W,), jnp.int32),
        pltpu.VMEM((TCH, H), jnp.float32),
        pltpu.VMEM((TCH, H), jnp.float32),
        pltpu.SemaphoreType.DMA,
    ],
)
def _emb(x_hbm, wtab_hbm, pos_hbm, out_hbm, idx_v, pos_v, rows_v, sem):
    wid = lax.axis_index("s") * NC + lax.axis_index("c")
    base = wid * ROWS_W
    pltpu.sync_copy(x_hbm.at[pl.ds(base, ROWS_W)], idx_v)
    for tc in range(NTC):
        pltpu.sync_copy(pos_hbm.at[pl.ds(tc * TCH, TCH)], pos_v)

        @pl.loop(0, SEQ_W)
        def _seq(bl):
            loc = bl * T + tc * TCH
            pltpu.async_copy(
                wtab_hbm.at[idx_v.at[pl.ds(loc, TCH)]], rows_v, sem
            ).wait()

            @pl.loop(0, TCH)
            def _row(r):
                for c in range(VPR):
                    sl = pl.ds(c * 16, 16)
                    plsc.addupdate(rows_v.at[r, sl], pos_v[r, sl])

            pltpu.sync_copy(rows_v, out_hbm.at[pl.ds(base + loc, TCH)])


def kernel(x, word_embeddings, position_embeddings):
    xf = x.reshape(N)
    pos2 = lax.slice_in_dim(position_embeddings, POS_OFF, POS_OFF + T, axis=0)
    out = _emb(xf, word_embeddings, pos2)
    return out.reshape(B, T, H)
